# baseline (speedup 1.0000x reference)
import functools

import jax
import jax.numpy as jnp
from jax import lax
from jax.experimental import pallas as pl
from jax.experimental.pallas import tpu as pltpu
from jax.experimental.pallas import tpu_sc as plsc

N = 10000
H = 128
E = 320000
ROUNDS = 3

NC = 2
NS = 16
NW = NC * NS
K = 125
EPW = E // NW
NCH = EPW // K
NP = 10240
RPT = NP // NS
NH = 2
CPH = NCH // NH


@functools.cache
def _make_sc_edge_aggregate():
    mesh = plsc.VectorSubcoreMesh(core_axis_name="c", subcore_axis_name="s")

    @functools.partial(
        pl.kernel,
        out_type=jax.ShapeDtypeStruct((NC, NP, H), jnp.float32),
        mesh=mesh,
        scratch_types=[
            pltpu.VMEM((CPH, K), jnp.int32),
            pltpu.VMEM((CPH, K), jnp.int32),
            pltpu.VMEM((K, H), jnp.float32),
            pltpu.VMEM((K, H), jnp.float32),
            pltpu.VMEM_SHARED((NP, H), jnp.float32),
            pltpu.SemaphoreType.DMA,
            pltpu.SemaphoreType.DMA,
        ],
    )
    def sc_edge_aggregate(msg_hbm, src_hbm, dst_hbm, zero_hbm, out_hbm,
                          src_v, dst_v, rows0, rows1, acc_sh, sem0, sem1):
        cid = lax.axis_index("c")
        sid = lax.axis_index("s")
        wid = cid * NS + sid

        def gather(j, rows, sem):
            return pltpu.make_async_copy(msg_hbm.at[src_v.at[j]], rows, sem)

        pltpu.sync_copy(zero_hbm, acc_sh.at[pl.ds(sid * RPT, RPT)])
        plsc.subcore_barrier()

        def two_chunks(i, carry):
            j0 = 2 * i
            gather(j0 + 1, rows1, sem1).start()
            gather(j0, rows0, sem0).wait()
            pltpu.sync_copy(rows0, acc_sh.at[dst_v.at[j0]], add=True)

            @pl.when(j0 + 2 < CPH)
            def _():
                gather(j0 + 2, rows0, sem0).start()

            gather(j0 + 1, rows1, sem1).wait()
            pltpu.sync_copy(rows1, acc_sh.at[dst_v.at[j0 + 1]], add=True)
            return carry

        for h in range(NH):
            base = wid * NCH + h * CPH
            pltpu.sync_copy(src_hbm.at[pl.ds(base, CPH)], src_v)
            pltpu.sync_copy(dst_hbm.at[pl.ds(base, CPH)], dst_v)
            gather(0, rows0, sem0).start()
            lax.fori_loop(0, CPH // 2, two_chunks, 0)
        plsc.subcore_barrier()
        pltpu.sync_copy(acc_sh.at[pl.ds(sid * RPT, RPT)],
                        out_hbm.at[cid, pl.ds(sid * RPT, RPT)])

    return sc_edge_aggregate


BM = 2000
_ROW = pl.BlockSpec((BM, H), lambda i: (i, 0))


def _wr(r):
    return pl.BlockSpec((1, H, H), lambda i: (r, 0, 0))


def _br(r):
    return pl.BlockSpec((1, 1, H), lambda i: (r, 0, 0))


def _pr(c):
    return pl.BlockSpec((1, BM, H), lambda i, c=c: (c, i, 0))


def _relu_mm(x, w, b):
    return jnp.maximum(
        jnp.dot(x, w, preferred_element_type=jnp.float32) + b, 0.0)


def _in_msg_body(z_ref, wi_ref, bi_ref, wm_ref, bm_ref, s_ref, m_ref):
    s = _relu_mm(z_ref[...], wi_ref[...], bi_ref[...])
    s_ref[...] = s
    m_ref[...] = _relu_mm(s, wm_ref[0], bm_ref[0])


def _in_msg(z, wi, bi, wm, bm):
    return pl.pallas_call(
        _in_msg_body,
        grid=(N // BM,),
        in_specs=[_ROW,
                  pl.BlockSpec((H, H), lambda i: (0, 0)),
                  pl.BlockSpec((1, H), lambda i: (0, 0)),
                  _wr(0), _br(0)],
        out_specs=[_ROW, _ROW],
        out_shape=[jax.ShapeDtypeStruct((N, H), jnp.float32)] * 2,
    )(z, wi, bi.reshape(1, H), wm, bm.reshape(ROUNDS, 1, H))


def _upd_msg_body(s_ref, p0_ref, p1_ref, wu_ref, bu_ref, wm_ref, bm_ref,
                  s_out_ref, m_ref):
    agg = p0_ref[0] + p1_ref[0]
    s = s_ref[...] + _relu_mm(agg, wu_ref[0], bu_ref[0])
    s_out_ref[...] = s
    m_ref[...] = _relu_mm(s, wm_ref[0], bm_ref[0])


def _upd_msg(s, p, wu, bu, wm, bm, r):
    return pl.pallas_call(
        _upd_msg_body,
        grid=(N // BM,),
        in_specs=[_ROW, _pr(0), _pr(1), _wr(r), _br(r), _wr(r + 1), _br(r + 1)],
        out_specs=[_ROW, _ROW],
        out_shape=[jax.ShapeDtypeStruct((N, H), jnp.float32)] * 2,
    )(s, p, p, wu, bu.reshape(ROUNDS, 1, H), wm, bm.reshape(ROUNDS, 1, H))


def _upd_out_body(s_ref, p0_ref, p1_ref, wu_ref, bu_ref, wo_ref, bo_ref,
                  o_ref):
    agg = p0_ref[0] + p1_ref[0]
    s = s_ref[...] + _relu_mm(agg, wu_ref[0], bu_ref[0])
    o_ref[...] = (
        jnp.dot(s, wo_ref[...], preferred_element_type=jnp.float32)
        + bo_ref[...])


def _upd_out(s, p, wu, bu, wo, bo, r, feat):
    return pl.pallas_call(
        _upd_out_body,
        grid=(N // BM,),
        in_specs=[_ROW, _pr(0), _pr(1), _wr(r), _br(r),
                  pl.BlockSpec((H, feat), lambda i: (0, 0)),
                  pl.BlockSpec((1, feat), lambda i: (0, 0))],
        out_specs=pl.BlockSpec((BM, feat), lambda i: (i, 0)),
        out_shape=jax.ShapeDtypeStruct((N, feat), jnp.float32),
    )(s, p, p, wu, bu.reshape(ROUNDS, 1, H), wo, bo.reshape(1, feat))


def kernel(z, edge_index, W_in, b_in, W_msg, b_msg, W_upd, b_upd, W_out, b_out):
    src = edge_index[0].reshape(E // K, K)
    dst = edge_index[1].reshape(E // K, K)
    zeros = jnp.zeros((RPT, H), jnp.float32)
    sc = _make_sc_edge_aggregate()

    state, message = _in_msg(z, W_in, b_in, W_msg, b_msg)
    for r in range(ROUNDS - 1):
        p = sc(message, src, dst, zeros)
        state, message = _upd_msg(state, p, W_upd, b_upd, W_msg, b_msg, r)
    p = sc(message, src, dst, zeros)
    return _upd_out(state, p, W_upd, b_upd, W_out, b_out, ROUNDS - 1,
                    W_out.shape[1])


# speedup vs baseline: 1.0338x; 1.0338x over previous
import functools

import jax
import jax.numpy as jnp
from jax import lax
from jax.experimental import pallas as pl
from jax.experimental.pallas import tpu as pltpu
from jax.experimental.pallas import tpu_sc as plsc

N = 10000
H = 128
E = 320000
ROUNDS = 3

NC = 2
NS = 16
NW = NC * NS
K = 125
EPW = E // NW
NCH = EPW // K
NP = 10240
RPT = NP // NS
NH = 2
CPH = NCH // NH


@functools.cache
def _make_sc_edge_aggregate():
    mesh = plsc.VectorSubcoreMesh(core_axis_name="c", subcore_axis_name="s")

    @functools.partial(
        pl.kernel,
        out_type=jax.ShapeDtypeStruct((NC, NP, H), jnp.float32),
        mesh=mesh,
        scratch_types=[
            pltpu.VMEM((CPH, K), jnp.int32),
            pltpu.VMEM((CPH, K), jnp.int32),
            pltpu.VMEM((K, H), jnp.float32),
            pltpu.VMEM((K, H), jnp.float32),
            pltpu.VMEM_SHARED((NP, H), jnp.float32),
            pltpu.SemaphoreType.DMA,
            pltpu.SemaphoreType.DMA,
        ],
    )
    def sc_edge_aggregate(msg_hbm, eidx_hbm, zero_hbm, out_hbm,
                          src_v, dst_v, rows0, rows1, acc_sh, sem0, sem1):
        cid = lax.axis_index("c")
        sid = lax.axis_index("s")
        wid = cid * NS + sid

        def gather(j, rows, sem):
            return pltpu.make_async_copy(msg_hbm.at[src_v.at[j]], rows, sem)

        pltpu.sync_copy(zero_hbm.at[pl.ds(sid * RPT, RPT)],
                        acc_sh.at[pl.ds(sid * RPT, RPT)])
        plsc.subcore_barrier()

        def two_chunks(i, carry):
            j0 = 2 * i
            gather(j0 + 1, rows1, sem1).start()
            gather(j0, rows0, sem0).wait()
            pltpu.sync_copy(rows0, acc_sh.at[dst_v.at[j0]], add=True)

            @pl.when(j0 + 2 < CPH)
            def _():
                gather(j0 + 2, rows0, sem0).start()

            gather(j0 + 1, rows1, sem1).wait()
            pltpu.sync_copy(rows1, acc_sh.at[dst_v.at[j0 + 1]], add=True)
            return carry

        for h in range(NH):
            base = wid * NCH + h * CPH
            pltpu.sync_copy(eidx_hbm.at[0, pl.ds(base, CPH)], src_v)
            pltpu.sync_copy(eidx_hbm.at[1, pl.ds(base, CPH)], dst_v)
            gather(0, rows0, sem0).start()
            lax.fori_loop(0, CPH // 2, two_chunks, 0)
        plsc.subcore_barrier()
        pltpu.sync_copy(acc_sh.at[pl.ds(sid * RPT, RPT)],
                        out_hbm.at[cid, pl.ds(sid * RPT, RPT)])

    return sc_edge_aggregate


BM = 2000
_ROW = pl.BlockSpec((BM, H), lambda i: (i, 0))


def _wr(r):
    return pl.BlockSpec((1, H, H), lambda i: (r, 0, 0))


def _br(r):
    return pl.BlockSpec((1, 1, H), lambda i: (r, 0, 0))


def _pr(c):
    return pl.BlockSpec((1, BM, H), lambda i, c=c: (c, i, 0))


def _relu_mm(x, w, b):
    return jnp.maximum(
        jnp.dot(x, w, preferred_element_type=jnp.float32) + b, 0.0)


def _in_msg_body(z_ref, wi_ref, bi_ref, wm_ref, bm_ref, s_ref, m_ref):
    s = _relu_mm(z_ref[...], wi_ref[...], bi_ref[...])
    s_ref[...] = s
    m_ref[...] = _relu_mm(s, wm_ref[0], bm_ref[0])


def _in_msg(z, wi, bi, wm, bm):
    return pl.pallas_call(
        _in_msg_body,
        grid=(N // BM,),
        in_specs=[_ROW,
                  pl.BlockSpec((H, H), lambda i: (0, 0)),
                  pl.BlockSpec((1, H), lambda i: (0, 0)),
                  _wr(0), _br(0)],
        out_specs=[_ROW, _ROW],
        out_shape=[jax.ShapeDtypeStruct((N, H), jnp.float32)] * 2,
    )(z, wi, bi.reshape(1, H), wm, bm.reshape(ROUNDS, 1, H))


def _upd_msg_body(s_ref, p0_ref, p1_ref, wu_ref, bu_ref, wm_ref, bm_ref,
                  s_out_ref, m_ref):
    agg = p0_ref[0] + p1_ref[0]
    s = s_ref[...] + _relu_mm(agg, wu_ref[0], bu_ref[0])
    s_out_ref[...] = s
    m_ref[...] = _relu_mm(s, wm_ref[0], bm_ref[0])


def _upd_msg(s, p, wu, bu, wm, bm, r):
    return pl.pallas_call(
        _upd_msg_body,
        grid=(N // BM,),
        in_specs=[_ROW, _pr(0), _pr(1), _wr(r), _br(r), _wr(r + 1), _br(r + 1)],
        out_specs=[_ROW, _ROW],
        out_shape=[jax.ShapeDtypeStruct((N, H), jnp.float32)] * 2,
    )(s, p, p, wu, bu.reshape(ROUNDS, 1, H), wm, bm.reshape(ROUNDS, 1, H))


def _upd_out_body(s_ref, p0_ref, p1_ref, wu_ref, bu_ref, wo_ref, bo_ref,
                  o_ref):
    agg = p0_ref[0] + p1_ref[0]
    s = s_ref[...] + _relu_mm(agg, wu_ref[0], bu_ref[0])
    o_ref[...] = (
        jnp.dot(s, wo_ref[...], preferred_element_type=jnp.float32)
        + bo_ref[...])


def _upd_out(s, p, wu, bu, wo, bo, r, feat):
    return pl.pallas_call(
        _upd_out_body,
        grid=(N // BM,),
        in_specs=[_ROW, _pr(0), _pr(1), _wr(r), _br(r),
                  pl.BlockSpec((H, feat), lambda i: (0, 0)),
                  pl.BlockSpec((1, feat), lambda i: (0, 0))],
        out_specs=pl.BlockSpec((BM, feat), lambda i: (i, 0)),
        out_shape=jax.ShapeDtypeStruct((N, feat), jnp.float32),
    )(s, p, p, wu, bu.reshape(ROUNDS, 1, H), wo, bo.reshape(1, feat))


def kernel(z, edge_index, W_in, b_in, W_msg, b_msg, W_upd, b_upd, W_out, b_out):
    eidx = edge_index.reshape(2, E // K, K)
    zeros = jnp.zeros((NP, H), jnp.float32)
    sc = _make_sc_edge_aggregate()

    state, message = _in_msg(z, W_in, b_in, W_msg, b_msg)
    for r in range(ROUNDS - 1):
        p = sc(message, eidx, zeros)
        state, message = _upd_msg(state, p, W_upd, b_upd, W_msg, b_msg, r)
    p = sc(message, eidx, zeros)
    return _upd_out(state, p, W_upd, b_upd, W_out, b_out, ROUNDS - 1,
                    W_out.shape[1])


# speedup vs baseline: 1.0527x; 1.0183x over previous
import functools

import jax
import jax.numpy as jnp
from jax import lax
from jax.experimental import pallas as pl
from jax.experimental.pallas import tpu as pltpu
from jax.experimental.pallas import tpu_sc as plsc

N = 10000
H = 128
E = 320000
ROUNDS = 3

NC = 2
NS = 16
NW = NC * NS
K = 125
EPW = E // NW
NCH = EPW // K
NP = 10240
RPT = NP // NS
NH = 2
CPH = NCH // NH


@functools.cache
def _make_sc_edge_aggregate():
    mesh = plsc.VectorSubcoreMesh(core_axis_name="c", subcore_axis_name="s")

    @functools.partial(
        pl.kernel,
        out_type=jax.ShapeDtypeStruct((NC, NP, H), jnp.float32),
        mesh=mesh,
        scratch_types=[
            pltpu.VMEM((CPH, K), jnp.int32),
            pltpu.VMEM((CPH, K), jnp.int32),
            pltpu.VMEM((K, H), jnp.float32),
            pltpu.VMEM((K, H), jnp.float32),
            pltpu.VMEM_SHARED((NP, H), jnp.float32),
            pltpu.SemaphoreType.DMA,
            pltpu.SemaphoreType.DMA,
        ],
    )
    def sc_edge_aggregate(msg_hbm, eidx_hbm, zero_hbm, out_hbm,
                          src_v, dst_v, rows0, rows1, acc_sh, sem0, sem1):
        cid = lax.axis_index("c")
        sid = lax.axis_index("s")
        wid = cid * NS + sid

        def gather(j, rows, sem):
            return pltpu.make_async_copy(msg_hbm.at[src_v.at[j]], rows, sem)

        pltpu.sync_copy(zero_hbm.at[pl.ds(sid * RPT, RPT)],
                        acc_sh.at[pl.ds(sid * RPT, RPT)])
        plsc.subcore_barrier()

        def two_chunks(i, carry):
            j0 = 2 * i
            gather(j0 + 1, rows1, sem1).start()
            gather(j0, rows0, sem0).wait()
            pltpu.sync_copy(rows0, acc_sh.at[dst_v.at[j0]], add=True)

            @pl.when(j0 + 2 < CPH)
            def _():
                gather(j0 + 2, rows0, sem0).start()

            gather(j0 + 1, rows1, sem1).wait()
            pltpu.sync_copy(rows1, acc_sh.at[dst_v.at[j0 + 1]], add=True)
            return carry

        for h in range(NH):
            base = wid * NCH + h * CPH
            pltpu.sync_copy(eidx_hbm.at[0, pl.ds(base, CPH)], src_v)
            pltpu.sync_copy(eidx_hbm.at[1, pl.ds(base, CPH)], dst_v)
            gather(0, rows0, sem0).start()
            lax.fori_loop(0, CPH // 2, two_chunks, 0)
        plsc.subcore_barrier()
        pltpu.sync_copy(acc_sh.at[pl.ds(sid * RPT, RPT)],
                        out_hbm.at[cid, pl.ds(sid * RPT, RPT)])

    return sc_edge_aggregate


BM = 5000
_ROW = pl.BlockSpec((BM, H), lambda i: (i, 0))


def _wr(r):
    return pl.BlockSpec((1, H, H), lambda i: (r, 0, 0))


def _br(r):
    return pl.BlockSpec((1, 1, H), lambda i: (r, 0, 0))


def _pr(c):
    return pl.BlockSpec((1, BM, H), lambda i, c=c: (c, i, 0))


def _relu_mm(x, w, b):
    return jnp.maximum(
        jnp.dot(x, w, preferred_element_type=jnp.float32) + b, 0.0)


def _in_msg_body(z_ref, wi_ref, bi_ref, wm_ref, bm_ref, s_ref, m_ref):
    s = _relu_mm(z_ref[...], wi_ref[...], bi_ref[...])
    s_ref[...] = s
    m_ref[...] = _relu_mm(s, wm_ref[0], bm_ref[0])


def _in_msg(z, wi, bi, wm, bm):
    return pl.pallas_call(
        _in_msg_body,
        grid=(N // BM,),
        in_specs=[_ROW,
                  pl.BlockSpec((H, H), lambda i: (0, 0)),
                  pl.BlockSpec((1, H), lambda i: (0, 0)),
                  _wr(0), _br(0)],
        out_specs=[_ROW, _ROW],
        out_shape=[jax.ShapeDtypeStruct((N, H), jnp.float32)] * 2,
    )(z, wi, bi.reshape(1, H), wm, bm.reshape(ROUNDS, 1, H))


def _upd_msg_body(s_ref, p0_ref, p1_ref, wu_ref, bu_ref, wm_ref, bm_ref,
                  s_out_ref, m_ref):
    agg = p0_ref[0] + p1_ref[0]
    s = s_ref[...] + _relu_mm(agg, wu_ref[0], bu_ref[0])
    s_out_ref[...] = s
    m_ref[...] = _relu_mm(s, wm_ref[0], bm_ref[0])


def _upd_msg(s, p, wu, bu, wm, bm, r):
    return pl.pallas_call(
        _upd_msg_body,
        grid=(N // BM,),
        in_specs=[_ROW, _pr(0), _pr(1), _wr(r), _br(r), _wr(r + 1), _br(r + 1)],
        out_specs=[_ROW, _ROW],
        out_shape=[jax.ShapeDtypeStruct((N, H), jnp.float32)] * 2,
    )(s, p, p, wu, bu.reshape(ROUNDS, 1, H), wm, bm.reshape(ROUNDS, 1, H))


def _upd_out_body(s_ref, p0_ref, p1_ref, wu_ref, bu_ref, wo_ref, bo_ref,
                  o_ref):
    agg = p0_ref[0] + p1_ref[0]
    s = s_ref[...] + _relu_mm(agg, wu_ref[0], bu_ref[0])
    o_ref[...] = (
        jnp.dot(s, wo_ref[...], preferred_element_type=jnp.float32)
        + bo_ref[...])


def _upd_out(s, p, wu, bu, wo, bo, r, feat):
    return pl.pallas_call(
        _upd_out_body,
        grid=(N // BM,),
        in_specs=[_ROW, _pr(0), _pr(1), _wr(r), _br(r),
                  pl.BlockSpec((H, feat), lambda i: (0, 0)),
                  pl.BlockSpec((1, feat), lambda i: (0, 0))],
        out_specs=pl.BlockSpec((BM, feat), lambda i: (i, 0)),
        out_shape=jax.ShapeDtypeStruct((N, feat), jnp.float32),
    )(s, p, p, wu, bu.reshape(ROUNDS, 1, H), wo, bo.reshape(1, feat))


def kernel(z, edge_index, W_in, b_in, W_msg, b_msg, W_upd, b_upd, W_out, b_out):
    eidx = edge_index.reshape(2, E // K, K)
    zeros = jnp.zeros((NP, H), jnp.float32)
    sc = _make_sc_edge_aggregate()

    state, message = _in_msg(z, W_in, b_in, W_msg, b_msg)
    for r in range(ROUNDS - 1):
        p = sc(message, eidx, zeros)
        state, message = _upd_msg(state, p, W_upd, b_upd, W_msg, b_msg, r)
    p = sc(message, eidx, zeros)
    return _upd_out(state, p, W_upd, b_upd, W_out, b_out, ROUNDS - 1,
                    W_out.shape[1])
